# baseline (speedup 1.0000x reference)
import jax
import jax.numpy as jnp
from jax.experimental import pallas as pl
from jax.experimental.pallas import tpu as pltpu

_N = 100
_IN = 128
_OUT = 128
_E = 16
_B_BLK = 128


def _combine_kernel(gw_ref, experts_ref, biases_ref, cw_ref, cb_ref):
    g = jax.nn.softmax(gw_ref[...], axis=-1)
    cw_ref[...] = jnp.dot(g, experts_ref[...], preferred_element_type=jnp.float32)
    cb_ref[...] = jnp.dot(g, biases_ref[...], preferred_element_type=jnp.float32)


def _apply_kernel(x_ref, cw_ref, cb_ref, out_ref):
    for n in range(_N):
        xn = x_ref[:, n * _IN : (n + 1) * _IN]
        wn = cw_ref[n]
        yn = jnp.dot(xn, wn, preferred_element_type=jnp.float32)
        out_ref[:, n * _OUT : (n + 1) * _OUT] = yn + cb_ref[:, n * _OUT : (n + 1) * _OUT]


def kernel(x, gate_weights, experts, expert_biases):
    batch = x.shape[0]
    experts2 = experts.reshape(_E, _IN * _OUT)

    cw2, cb = pl.pallas_call(
        _combine_kernel,
        out_shape=(
            jax.ShapeDtypeStruct((_N, _IN * _OUT), jnp.float32),
            jax.ShapeDtypeStruct((_N, _OUT), jnp.float32),
        ),
    )(gate_weights, experts2, expert_biases)
    cw = cw2.reshape(_N, _IN, _OUT)
    cb2 = cb.reshape(1, _N * _OUT)
    x2 = x.reshape(batch, _N * _IN)

    grid = (batch // _B_BLK,)
    out2 = pl.pallas_call(
        _apply_kernel,
        grid=grid,
        in_specs=[
            pl.BlockSpec((_B_BLK, _N * _IN), lambda i: (i, 0)),
            pl.BlockSpec((_N, _IN, _OUT), lambda i: (0, 0, 0)),
            pl.BlockSpec((1, _N * _OUT), lambda i: (0, 0)),
        ],
        out_specs=pl.BlockSpec((_B_BLK, _N * _OUT), lambda i: (i, 0)),
        out_shape=jax.ShapeDtypeStruct((batch, _N * _OUT), jnp.float32),
        compiler_params=pltpu.CompilerParams(
            dimension_semantics=("arbitrary",),
        ),
    )(x2, cw, cb2)
    return out2.reshape(batch, _N, _OUT)


# speedup vs baseline: 1.3859x; 1.3859x over previous
import jax
import jax.numpy as jnp
from jax.experimental import pallas as pl
from jax.experimental.pallas import tpu as pltpu

_N = 100
_IN = 128
_OUT = 128
_E = 16
_CORES = 2


def _combine_kernel(gw_ref, experts_ref, biases_ref, cw_ref, cb_ref):
    g = jax.nn.softmax(gw_ref[...], axis=-1)
    cw = jnp.dot(g, experts_ref[...], preferred_element_type=jnp.float32)
    cw_ref[...] = cw.astype(jnp.bfloat16)
    cb_ref[...] = jnp.dot(g, biases_ref[...], preferred_element_type=jnp.float32)


def _apply_kernel(x_hbm, w_ref, b_ref, out_hbm, xbuf, ybuf, in_sem, out_sem):
    core = pl.program_id(0)
    n = pl.program_id(1)
    half = x_hbm.shape[0] // _CORES
    b0 = core * half

    def in_copy(step, slot):
        return pltpu.make_async_copy(
            x_hbm.at[pl.ds(b0, half), step, :], xbuf.at[slot], in_sem.at[slot]
        )

    def out_copy(step, slot):
        return pltpu.make_async_copy(
            ybuf.at[slot], out_hbm.at[pl.ds(b0, half), step, :], out_sem.at[slot]
        )

    slot = jax.lax.rem(n, 2)
    nxt = 1 - slot

    @pl.when(n == 0)
    def _prologue():
        in_copy(0, 0).start()

    in_copy(n, slot).wait()

    @pl.when(n + 1 < _N)
    def _prefetch():
        in_copy(n + 1, nxt).start()

    @pl.when(n >= 2)
    def _drain():
        out_copy(n - 2, slot).wait()

    xb = xbuf[slot].astype(jnp.bfloat16)
    y = jnp.dot(xb, w_ref[n], preferred_element_type=jnp.float32)
    ybuf[slot] = y + b_ref[pl.ds(n, 1), :]
    out_copy(n, slot).start()

    @pl.when(n == _N - 1)
    def _epilogue():
        out_copy(n - 1, nxt).wait()
        out_copy(n, slot).wait()


def kernel(x, gate_weights, experts, expert_biases):
    batch = x.shape[0]
    half = batch // _CORES
    experts2 = experts.reshape(_E, _IN * _OUT)

    cw2, cb = pl.pallas_call(
        _combine_kernel,
        out_shape=(
            jax.ShapeDtypeStruct((_N, _IN * _OUT), jnp.bfloat16),
            jax.ShapeDtypeStruct((_N, _OUT), jnp.float32),
        ),
    )(gate_weights, experts2, expert_biases)
    cw = cw2.reshape(_N, _IN, _OUT)

    out = pl.pallas_call(
        _apply_kernel,
        grid=(_CORES, _N),
        in_specs=[
            pl.BlockSpec(memory_space=pltpu.MemorySpace.HBM),
            pl.BlockSpec((_N, _IN, _OUT), lambda c, n: (0, 0, 0)),
            pl.BlockSpec((_N, _OUT), lambda c, n: (0, 0)),
        ],
        out_specs=pl.BlockSpec(memory_space=pltpu.MemorySpace.HBM),
        out_shape=jax.ShapeDtypeStruct((batch, _N, _OUT), jnp.float32),
        scratch_shapes=[
            pltpu.MemorySpace.VMEM((2, half, _IN), jnp.float32),
            pltpu.MemorySpace.VMEM((2, half, _OUT), jnp.float32),
            pltpu.SemaphoreType.DMA((2,)),
            pltpu.SemaphoreType.DMA((2,)),
        ],
        compiler_params=pltpu.CompilerParams(
            dimension_semantics=("parallel", "arbitrary"),
        ),
    )(x, cw, cb)
    return out


# speedup vs baseline: 1.4791x; 1.0673x over previous
import jax
import jax.numpy as jnp
from jax.experimental import pallas as pl
from jax.experimental.pallas import tpu as pltpu

_N = 100
_IN = 128
_OUT = 128
_E = 16
_B_BLK = 128


def _combine_kernel(gw_ref, experts_ref, biases_ref, cw_ref, cb_ref):
    g = jax.nn.softmax(gw_ref[...], axis=-1)
    cw = jnp.dot(g, experts_ref[...], preferred_element_type=jnp.float32)
    cw_ref[...] = cw.astype(jnp.bfloat16)
    cb_ref[...] = jnp.dot(g, biases_ref[...], preferred_element_type=jnp.float32)


def _apply_kernel(x_ref, w_ref, b_ref, out_ref):
    xb = x_ref[...].astype(jnp.bfloat16)
    for n in range(_N):
        y = jnp.dot(xb[:, n, :], w_ref[n], preferred_element_type=jnp.float32)
        out_ref[:, n, :] = y + b_ref[n : n + 1, :]


def kernel(x, gate_weights, experts, expert_biases):
    batch = x.shape[0]
    experts2 = experts.reshape(_E, _IN * _OUT)

    cw2, cb = pl.pallas_call(
        _combine_kernel,
        out_shape=(
            jax.ShapeDtypeStruct((_N, _IN * _OUT), jnp.bfloat16),
            jax.ShapeDtypeStruct((_N, _OUT), jnp.float32),
        ),
    )(gate_weights, experts2, expert_biases)
    cw = cw2.reshape(_N, _IN, _OUT)

    out = pl.pallas_call(
        _apply_kernel,
        grid=(batch // _B_BLK,),
        in_specs=[
            pl.BlockSpec((_B_BLK, _N, _IN), lambda i: (i, 0, 0)),
            pl.BlockSpec((_N, _IN, _OUT), lambda i: (0, 0, 0)),
            pl.BlockSpec((_N, _OUT), lambda i: (0, 0)),
        ],
        out_specs=pl.BlockSpec((_B_BLK, _N, _OUT), lambda i: (i, 0, 0)),
        out_shape=jax.ShapeDtypeStruct((batch, _N, _OUT), jnp.float32),
        compiler_params=pltpu.CompilerParams(
            dimension_semantics=("parallel",),
        ),
    )(x, cw, cb)
    return out


# speedup vs baseline: 1.5113x; 1.0217x over previous
import jax
import jax.numpy as jnp
from jax.experimental import pallas as pl
from jax.experimental.pallas import tpu as pltpu

_N = 100
_IN = 128
_OUT = 128
_E = 16
_B_BLK = 512


def _combine_kernel(gw_ref, experts_ref, biases_ref, cw_ref, cb_ref):
    g = jax.nn.softmax(gw_ref[...], axis=-1)
    cw = jnp.dot(g, experts_ref[...], preferred_element_type=jnp.float32)
    cw_ref[...] = cw.astype(jnp.bfloat16)
    cb_ref[...] = jnp.dot(g, biases_ref[...], preferred_element_type=jnp.float32)


def _apply_kernel(x_ref, w_ref, b_ref, out_ref):
    xb = x_ref[0].astype(jnp.bfloat16)
    y = jnp.dot(xb, w_ref[0], preferred_element_type=jnp.float32)
    out_ref[0] = y + b_ref[0]


def kernel(x, gate_weights, experts, expert_biases):
    batch = x.shape[0]
    experts2 = experts.reshape(_E, _IN * _OUT)

    cw2, cb = pl.pallas_call(
        _combine_kernel,
        out_shape=(
            jax.ShapeDtypeStruct((_N, _IN * _OUT), jnp.bfloat16),
            jax.ShapeDtypeStruct((_N, _OUT), jnp.float32),
        ),
    )(gate_weights, experts2, expert_biases)
    cw = cw2.reshape(_N, _IN, _OUT)
    cb3 = cb.reshape(_N, 1, _OUT)

    xt = jnp.swapaxes(x, 0, 1)
    out_t = pl.pallas_call(
        _apply_kernel,
        grid=(_N, batch // _B_BLK),
        in_specs=[
            pl.BlockSpec((1, _B_BLK, _IN), lambda n, j: (n, j, 0)),
            pl.BlockSpec((1, _IN, _OUT), lambda n, j: (n, 0, 0)),
            pl.BlockSpec((1, 1, _OUT), lambda n, j: (n, 0, 0)),
        ],
        out_specs=pl.BlockSpec((1, _B_BLK, _OUT), lambda n, j: (n, j, 0)),
        out_shape=jax.ShapeDtypeStruct((_N, batch, _OUT), jnp.float32),
        compiler_params=pltpu.CompilerParams(
            dimension_semantics=("parallel", "parallel"),
        ),
    )(xt, cw, cb3)
    return jnp.swapaxes(out_t, 0, 1)


# speedup vs baseline: 5.6879x; 3.7636x over previous
import jax
import jax.numpy as jnp
from jax.experimental import pallas as pl
from jax.experimental.pallas import tpu as pltpu

_N = 100
_IN = 128
_OUT = 128
_E = 16
_N_BLK = 4


def _combine_kernel(gw_ref, experts_ref, biases_ref, cw_ref, cb_ref):
    g = jax.nn.softmax(gw_ref[...], axis=-1)
    cw = jnp.dot(g, experts_ref[...], preferred_element_type=jnp.float32)
    cw_ref[...] = cw.astype(jnp.bfloat16)
    cb_ref[...] = jnp.dot(g, biases_ref[...], preferred_element_type=jnp.float32)


def _apply_kernel(x_ref, w_ref, b_ref, out_ref):
    for k in range(_N_BLK):
        xb = x_ref[k].astype(jnp.bfloat16)
        y = jnp.dot(xb, w_ref[k], preferred_element_type=jnp.float32)
        out_ref[k] = y + b_ref[k]


def kernel(x, gate_weights, experts, expert_biases):
    batch = x.shape[0]
    experts2 = experts.reshape(_E, _IN * _OUT)

    cw2, cb = pl.pallas_call(
        _combine_kernel,
        out_shape=(
            jax.ShapeDtypeStruct((_N, _IN * _OUT), jnp.bfloat16),
            jax.ShapeDtypeStruct((_N, _OUT), jnp.float32),
        ),
    )(gate_weights, experts2, expert_biases)
    cw = cw2.reshape(_N, _IN, _OUT)
    cb3 = cb.reshape(_N, 1, _OUT)

    xt = jnp.swapaxes(x, 0, 1)
    out_t = pl.pallas_call(
        _apply_kernel,
        grid=(_N // _N_BLK,),
        in_specs=[
            pl.BlockSpec((_N_BLK, batch, _IN), lambda n: (n, 0, 0)),
            pl.BlockSpec((_N_BLK, _IN, _OUT), lambda n: (n, 0, 0)),
            pl.BlockSpec((_N_BLK, 1, _OUT), lambda n: (n, 0, 0)),
        ],
        out_specs=pl.BlockSpec((_N_BLK, batch, _OUT), lambda n: (n, 0, 0)),
        out_shape=jax.ShapeDtypeStruct((_N, batch, _OUT), jnp.float32),
        compiler_params=pltpu.CompilerParams(
            dimension_semantics=("parallel",),
        ),
    )(xt, cw, cb3)
    return jnp.swapaxes(out_t, 0, 1)
